# baseline (speedup 1.0000x reference)
import functools

import jax
import jax.numpy as jnp
from jax import lax
from jax.experimental import pallas as pl
from jax.experimental.pallas import tpu as pltpu
from jax.experimental.pallas import tpu_sc as plsc

N = 10000
E = 320000
D_IN = 128
D_HID = 256
D_OUT = 1
NUM_GRAPHS = 16

NC = 2
NS = 16
NPAD = 10240
BLK = 2048
NB = NPAD // BLK
RPT = NPAD // NS
CHUNK = 128
EPAD = 327680
NQ = 4
NR = 2

_f32 = jnp.float32


def _sc_mesh():
    return plsc.VectorSubcoreMesh(core_axis_name="c", subcore_axis_name="s")


def _deg_sc(dst2d, zeros16, ones16):
    nch = EPAD // (NC * NS) // CHUNK

    @functools.partial(
        pl.kernel,
        out_type=jax.ShapeDtypeStruct((2 * NPAD, 16), _f32),
        mesh=_sc_mesh(),
        compiler_params=pltpu.CompilerParams(use_tc_tiling_on_sc=False),
        scratch_types=[
            pltpu.VMEM((NQ, CHUNK), jnp.int32),
            pltpu.VMEM((CHUNK, 16), _f32),
            pltpu.VMEM_SHARED((NPAD, 16), _f32),
            pltpu.SemaphoreType.DMA,
            pltpu.SemaphoreType.DMA,
            pltpu.SemaphoreType.DMA,
            pltpu.SemaphoreType.DMA,
        ],
    )
    def k(dst_hbm, z_hbm, ones_hbm, out_hbm, idd_v, ones_v, acc_sh,
          q0, q1, q2, q3):
        qs = (q0, q1, q2, q3)
        c = lax.axis_index("c")
        s = lax.axis_index("s")
        rows0 = s * RPT
        drow0 = (s * NC + c) * nch
        pltpu.sync_copy(z_hbm.at[pl.ds(rows0, RPT)], acc_sh.at[pl.ds(rows0, RPT)])
        pltpu.sync_copy(ones_hbm, ones_v)
        plsc.subcore_barrier()

        def issue(slot, row):
            pltpu.async_copy(dst_hbm.at[pl.ds(drow0 + row, 1)],
                             idd_v.at[pl.ds(slot, 1)], qs[slot])

        def wait(slot):
            pltpu.make_async_copy(dst_hbm.at[pl.ds(0, 1)],
                                  idd_v.at[pl.ds(slot, 1)], qs[slot]).wait()

        for b in range(NQ):
            issue(b, b)

        def body(g, carry):
            for b in range(NQ):
                i = g * NQ + b
                wait(b)
                pltpu.sync_copy(ones_v, acc_sh.at[idd_v.at[b]], add=True)
                issue(b, jnp.minimum(i + NQ, nch - 1))
            return carry

        lax.fori_loop(0, nch // NQ, body, 0)
        for b in range(NQ):
            wait(b)
        plsc.subcore_barrier()
        pltpu.sync_copy(acc_sh.at[pl.ds(rows0, RPT)],
                        out_hbm.at[pl.ds(c * NPAD + rows0, RPT)])

    return k(dst2d, zeros16, ones16)


def _agg_pipe(tab_hbm, src_hbm, dst_hbm, z_hbm, acc_sh, ids_v, idd_v, rows_v,
              iqs, gss, sss, srow0, drow0, nch, chunk):
    def issue_idx(slot, row):
        pltpu.async_copy(src_hbm.at[pl.ds(srow0 + row, 1)],
                         ids_v.at[pl.ds(slot, 1)], iqs[slot])
        pltpu.async_copy(dst_hbm.at[pl.ds(drow0 + row, 1)],
                         idd_v.at[pl.ds(slot, 1)], iqs[slot])

    def wait_idx(slot):
        pltpu.make_async_copy(src_hbm.at[pl.ds(0, 1)],
                              ids_v.at[pl.ds(slot, 1)], iqs[slot]).wait()
        pltpu.make_async_copy(dst_hbm.at[pl.ds(0, 1)],
                              idd_v.at[pl.ds(slot, 1)], iqs[slot]).wait()

    def issue_gather(slot, idx_slot):
        pltpu.async_copy(tab_hbm.at[ids_v.at[idx_slot]], rows_v.at[slot],
                         gss[slot])

    def wait_gather(slot):
        pltpu.make_async_copy(tab_hbm.at[pl.ds(0, chunk)], rows_v.at[slot],
                              gss[slot]).wait()

    def issue_scatter(slot, idx_slot):
        pltpu.async_copy(rows_v.at[slot], acc_sh.at[idd_v.at[idx_slot]],
                         sss[slot], add=True)

    def wait_scatter(slot):
        pltpu.make_async_copy(rows_v.at[slot], acc_sh.at[idd_v.at[0]],
                              sss[slot]).wait()

    for q in range(6):
        issue_idx(q, q)
    wait_idx(0)
    wait_idx(1)
    pltpu.sync_copy(z_hbm.at[pl.ds(0, chunk)], rows_v.at[2])
    pltpu.sync_copy(z_hbm.at[pl.ds(0, chunk)], rows_v.at[3])
    issue_scatter(2, 0)
    issue_scatter(3, 1)
    issue_gather(0, 0)
    issue_gather(1, 1)

    def body(g, carry):
        for b in range(8):
            i = g * 8 + b
            r = b % 4
            wait_gather(r)
            issue_scatter(r, b)
            wait_scatter((r + 2) % 4)
            wait_idx((b + 2) % 8)
            issue_gather((r + 2) % 4, (b + 2) % 8)
            issue_idx((b + 6) % 8, jnp.minimum(i + 6, nch - 1))
        return carry

    lax.fori_loop(0, nch // 8, body, 0)
    for q in (2, 3, 4, 5):
        wait_idx(q)
    for r in (0, 1):
        wait_gather(r)
    for r in (2, 3):
        wait_scatter(r)


def _agg128_sc(table, src2d2, dst2d, zeros128):
    ch = 64
    nch = EPAD // NS // ch

    @functools.partial(
        pl.kernel,
        out_type=jax.ShapeDtypeStruct((2 * NPAD, 128), _f32),
        mesh=_sc_mesh(),
        compiler_params=pltpu.CompilerParams(use_tc_tiling_on_sc=False),
        scratch_types=(
            [pltpu.VMEM((8, ch), jnp.int32),
             pltpu.VMEM((8, ch), jnp.int32),
             pltpu.VMEM((4, ch, 128), _f32),
             pltpu.VMEM_SHARED((NPAD, 128), _f32)]
            + [pltpu.SemaphoreType.DMA] * 16
        ),
    )
    def k(tab_hbm, src_hbm, dst_hbm, z_hbm, out_hbm, ids_v, idd_v, rows_v,
          acc_sh, *sems):
        c = lax.axis_index("c")
        s = lax.axis_index("s")
        rows0 = s * RPT
        pltpu.sync_copy(z_hbm.at[pl.ds(rows0, RPT)], acc_sh.at[pl.ds(rows0, RPT)])
        plsc.subcore_barrier()
        _agg_pipe(tab_hbm, src_hbm, dst_hbm, z_hbm, acc_sh, ids_v, idd_v,
                  rows_v, sems[:8], sems[8:12], sems[12:16],
                  c * (EPAD // ch) + s * nch, s * nch, nch, ch)
        plsc.subcore_barrier()
        pltpu.sync_copy(acc_sh.at[pl.ds(rows0, RPT)],
                        out_hbm.at[pl.ds(c * NPAD + rows0, RPT)])

    return k(table, src2d2, dst2d, zeros128)


def _agg16_sc(table, src2d, dst2d, zeros16):
    nch = EPAD // (NC * NS) // CHUNK

    @functools.partial(
        pl.kernel,
        out_type=jax.ShapeDtypeStruct((2 * NPAD, 16), _f32),
        mesh=_sc_mesh(),
        compiler_params=pltpu.CompilerParams(use_tc_tiling_on_sc=False),
        scratch_types=(
            [pltpu.VMEM((8, CHUNK), jnp.int32),
             pltpu.VMEM((8, CHUNK), jnp.int32),
             pltpu.VMEM((4, CHUNK, 16), _f32),
             pltpu.VMEM_SHARED((NPAD, 16), _f32)]
            + [pltpu.SemaphoreType.DMA] * 16
        ),
    )
    def k(tab_hbm, src_hbm, dst_hbm, z_hbm, out_hbm, ids_v, idd_v, rows_v,
          acc_sh, *sems):
        c = lax.axis_index("c")
        s = lax.axis_index("s")
        rows0 = s * RPT
        wid = s * NC + c
        pltpu.sync_copy(z_hbm.at[pl.ds(rows0, RPT)], acc_sh.at[pl.ds(rows0, RPT)])
        plsc.subcore_barrier()
        _agg_pipe(tab_hbm, src_hbm, dst_hbm, z_hbm, acc_sh, ids_v, idd_v,
                  rows_v, sems[:8], sems[8:12], sems[12:16],
                  wid * nch, wid * nch, nch, CHUNK)
        plsc.subcore_barrier()
        pltpu.sync_copy(acc_sh.at[pl.ds(rows0, RPT)],
                        out_hbm.at[pl.ds(c * NPAD + rows0, RPT)])

    return k(table, src2d, dst2d, zeros16)


def _k1_body(x_ref, w_ref, dega_ref, degb_ref, g1s_ref, dis_ref):
    deg = dega_ref[:, :1] + degb_ref[:, :1] + 1.0
    dis = lax.rsqrt(deg)
    g = jnp.dot(x_ref[...], w_ref[...], preferred_element_type=_f32)
    g1s_ref[...] = g * dis
    dis_ref[...] = jnp.broadcast_to(dis, (BLK, 16))


def _k1(xp, W1, deg2):
    return pl.pallas_call(
        _k1_body,
        grid=(NB, 2),
        in_specs=[
            pl.BlockSpec((BLK, 128), lambda i, j: (i, 0)),
            pl.BlockSpec((128, 128), lambda i, j: (0, j)),
            pl.BlockSpec((BLK, 16), lambda i, j: (i, 0)),
            pl.BlockSpec((BLK, 16), lambda i, j: (i + NB, 0)),
        ],
        out_specs=[
            pl.BlockSpec((BLK, 128), lambda i, j: (j * NB + i, 0)),
            pl.BlockSpec((BLK, 16), lambda i, j: (i, 0)),
        ],
        out_shape=[
            jax.ShapeDtypeStruct((2 * NPAD, 128), _f32),
            jax.ShapeDtypeStruct((NPAD, 16), _f32),
        ],
    )(xp, W1, deg2, deg2)


def _k2_body(slo_ref, shi_ref, glo_ref, ghi_ref, dis_ref, b_ref, wa_ref,
             wb_ref, out_ref):
    dis = dis_ref[:, :1]
    hlo = jnp.maximum(dis * (slo_ref[...] + glo_ref[...]) + b_ref[0:1, :], 0.0)
    hhi = jnp.maximum(dis * (shi_ref[...] + ghi_ref[...]) + b_ref[1:2, :], 0.0)
    g2 = (jnp.dot(hlo, wa_ref[...], preferred_element_type=_f32)
          + jnp.dot(hhi, wb_ref[...], preferred_element_type=_f32))
    out_ref[...] = dis * g2


def _k2(s1, g1s, dis, b1r, W2):
    return pl.pallas_call(
        _k2_body,
        grid=(NB, 2),
        in_specs=[
            pl.BlockSpec((BLK, 128), lambda i, j: (i, 0)),
            pl.BlockSpec((BLK, 128), lambda i, j: (i + NB, 0)),
            pl.BlockSpec((BLK, 128), lambda i, j: (i, 0)),
            pl.BlockSpec((BLK, 128), lambda i, j: (i + NB, 0)),
            pl.BlockSpec((BLK, 16), lambda i, j: (i, 0)),
            pl.BlockSpec((2, 128), lambda i, j: (0, 0)),
            pl.BlockSpec((128, 128), lambda i, j: (0, j)),
            pl.BlockSpec((128, 128), lambda i, j: (1, j)),
        ],
        out_specs=pl.BlockSpec((BLK, 128), lambda i, j: (j * NB + i, 0)),
        out_shape=jax.ShapeDtypeStruct((2 * NPAD, 128), _f32),
    )(s1, s1, g1s, g1s, dis, b1r, W2, W2)


def _k3_body(slo_ref, shi_ref, glo_ref, ghi_ref, dis_ref, b_ref, wa_ref,
             wb_ref, out_ref):
    dis = dis_ref[:, :1]
    hlo = jnp.maximum(dis * (slo_ref[...] + glo_ref[...]) + b_ref[0:1, :], 0.0)
    hhi = jnp.maximum(dis * (shi_ref[...] + ghi_ref[...]) + b_ref[1:2, :], 0.0)
    g3 = (jnp.dot(hlo, wa_ref[...], preferred_element_type=_f32)
          + jnp.dot(hhi, wb_ref[...], preferred_element_type=_f32))
    out_ref[...] = dis * g3


def _k3(s2, g2s, dis, b2r, W316):
    return pl.pallas_call(
        _k3_body,
        grid=(NB,),
        in_specs=[
            pl.BlockSpec((BLK, 128), lambda i: (i, 0)),
            pl.BlockSpec((BLK, 128), lambda i: (i + NB, 0)),
            pl.BlockSpec((BLK, 128), lambda i: (i, 0)),
            pl.BlockSpec((BLK, 128), lambda i: (i + NB, 0)),
            pl.BlockSpec((BLK, 16), lambda i: (i, 0)),
            pl.BlockSpec((2, 128), lambda i: (0, 0)),
            pl.BlockSpec((128, 16), lambda i: (0, 0)),
            pl.BlockSpec((128, 16), lambda i: (1, 0)),
        ],
        out_specs=pl.BlockSpec((BLK, 16), lambda i: (i, 0)),
        out_shape=jax.ShapeDtypeStruct((NPAD, 16), _f32),
    )(s2, s2, g2s, g2s, dis, b2r, W316, W316)


def _k4_body(s3a_ref, s3b_ref, g3s_ref, dis_ref, batch_ref, b3_ref, out_ref):
    s3 = s3a_ref[...] + s3b_ref[...]
    y = dis_ref[...] * (s3 + g3s_ref[...]) + b3_ref[...]
    gids = lax.broadcasted_iota(jnp.int32, (1, 16), 1)
    onehot = (batch_ref[...] == gids).astype(_f32)
    sums = jnp.sum(onehot * y, axis=0)
    counts = jnp.sum(onehot, axis=0)
    out_ref[...] = (sums / jnp.maximum(counts, 1.0))[None, :]


def _k4(s3, g3s16, dis, batchi, b3r):
    return pl.pallas_call(
        _k4_body,
        grid=(1,),
        in_specs=[
            pl.BlockSpec((NPAD, 16), lambda i: (0, 0)),
            pl.BlockSpec((NPAD, 16), lambda i: (1, 0)),
            pl.BlockSpec((NPAD, 16), lambda i: (0, 0)),
            pl.BlockSpec((NPAD, 16), lambda i: (0, 0)),
            pl.BlockSpec((NPAD, 16), lambda i: (0, 0)),
            pl.BlockSpec((1, 16), lambda i: (0, 0)),
        ],
        out_specs=pl.BlockSpec((1, 16), lambda i: (0, 0)),
        out_shape=jax.ShapeDtypeStruct((1, 16), _f32),
    )(s3, s3, g3s16, dis, batchi, b3r)


def kernel(x, edge_index, batch, W1, b1, W2, b2, W3, b3):
    pad = jnp.full((EPAD - E,), NPAD - 1, dtype=jnp.int32)
    srcp = jnp.concatenate([edge_index[0], pad])
    dstp = jnp.concatenate([edge_index[1], pad])
    src2d = srcp.reshape(EPAD // CHUNK, CHUNK)
    src2d2_64 = jnp.concatenate([srcp, srcp + NPAD]).reshape(2 * EPAD // 64,
                                                             64)
    dst2d = dstp.reshape(EPAD // CHUNK, CHUNK)
    dst2d_64 = dstp.reshape(EPAD // 64, 64)
    xp = jnp.pad(x, ((0, NPAD - N), (0, 0)))
    batchp = jnp.pad(batch, (0, NPAD - N), constant_values=NUM_GRAPHS)
    batchi = jnp.broadcast_to(batchp[:, None], (NPAD, 16))
    b1r = b1.reshape(2, 128)
    b2r = b2.reshape(2, 128)
    W316 = jnp.tile(W3, (1, 16))
    b3r = jnp.broadcast_to(b3.reshape(1, 1), (1, 16))
    zeros128 = jnp.zeros((NPAD, 128), _f32)
    zeros16 = jnp.zeros((NPAD, 16), _f32)
    ones16 = jnp.ones((CHUNK, 16), _f32)

    deg2 = _deg_sc(dst2d, zeros16, ones16)
    g1s, dis = _k1(xp, W1, deg2)
    s1 = _agg128_sc(g1s, src2d2_64, dst2d_64, zeros128)
    g2s = _k2(s1, g1s, dis, b1r, W2)
    s2 = _agg128_sc(g2s, src2d2_64, dst2d_64, zeros128)
    g3s16 = _k3(s2, g2s, dis, b2r, W316)
    s3 = _agg16_sc(g3s16, src2d, dst2d, zeros16)
    out = _k4(s3, g3s16, dis, batchi, b3r)
    return out.reshape(NUM_GRAPHS, 1)


# speedup vs baseline: 2.2430x; 2.2430x over previous
import functools

import jax
import jax.numpy as jnp
from jax import lax
from jax.experimental import pallas as pl
from jax.experimental.pallas import tpu as pltpu
from jax.experimental.pallas import tpu_sc as plsc

N = 10000
E = 320000
D_IN = 128
D_HID = 256
D_OUT = 1
NUM_GRAPHS = 16

NC = 2
NS = 16
NPAD = 10240
BLK = 2048
NB = NPAD // BLK
RPT = NPAD // NS
CHUNK = 128
EPAD = 327680
NQ = 4
NR = 2

_f32 = jnp.float32


def _sc_mesh():
    return plsc.VectorSubcoreMesh(core_axis_name="c", subcore_axis_name="s")


def _deg_sc(dst2d, zeros16, ones16):
    nch = EPAD // (NC * NS) // CHUNK

    @functools.partial(
        pl.kernel,
        out_type=jax.ShapeDtypeStruct((2 * NPAD, 16), _f32),
        mesh=_sc_mesh(),
        compiler_params=pltpu.CompilerParams(use_tc_tiling_on_sc=False),
        scratch_types=[
            pltpu.VMEM((NQ, CHUNK), jnp.int32),
            pltpu.VMEM((CHUNK, 16), _f32),
            pltpu.VMEM_SHARED((NPAD, 16), _f32),
            pltpu.SemaphoreType.DMA,
            pltpu.SemaphoreType.DMA,
            pltpu.SemaphoreType.DMA,
            pltpu.SemaphoreType.DMA,
        ],
    )
    def k(dst_hbm, z_hbm, ones_hbm, out_hbm, idd_v, ones_v, acc_sh,
          q0, q1, q2, q3):
        qs = (q0, q1, q2, q3)
        c = lax.axis_index("c")
        s = lax.axis_index("s")
        rows0 = s * RPT
        drow0 = (s * NC + c) * nch
        pltpu.sync_copy(z_hbm.at[pl.ds(rows0, RPT)], acc_sh.at[pl.ds(rows0, RPT)])
        pltpu.sync_copy(ones_hbm, ones_v)
        plsc.subcore_barrier()

        def issue(slot, row):
            pltpu.async_copy(dst_hbm.at[pl.ds(drow0 + row, 1)],
                             idd_v.at[pl.ds(slot, 1)], qs[slot])

        def wait(slot):
            pltpu.make_async_copy(dst_hbm.at[pl.ds(0, 1)],
                                  idd_v.at[pl.ds(slot, 1)], qs[slot]).wait()

        for b in range(NQ):
            issue(b, b)

        def body(g, carry):
            for b in range(NQ):
                i = g * NQ + b
                wait(b)
                pltpu.sync_copy(ones_v, acc_sh.at[idd_v.at[b]], add=True)
                issue(b, jnp.minimum(i + NQ, nch - 1))
            return carry

        lax.fori_loop(0, nch // NQ, body, 0)
        for b in range(NQ):
            wait(b)
        plsc.subcore_barrier()
        pltpu.sync_copy(acc_sh.at[pl.ds(rows0, RPT)],
                        out_hbm.at[pl.ds(c * NPAD + rows0, RPT)])

    return k(dst2d, zeros16, ones16)


def _agg_pipe(tab_hbm, src_hbm, dst_hbm, z_hbm, acc_sh, ids_v, idd_v, rows_v,
              iqs, gss, sss, srow0, drow0, nch, chunk):
    def issue_idx(slot, row):
        pltpu.async_copy(src_hbm.at[pl.ds(srow0 + row, 1)],
                         ids_v.at[pl.ds(slot, 1)], iqs[slot])
        pltpu.async_copy(dst_hbm.at[pl.ds(drow0 + row, 1)],
                         idd_v.at[pl.ds(slot, 1)], iqs[slot])

    def wait_idx(slot):
        pltpu.make_async_copy(src_hbm.at[pl.ds(0, 1)],
                              ids_v.at[pl.ds(slot, 1)], iqs[slot]).wait()
        pltpu.make_async_copy(dst_hbm.at[pl.ds(0, 1)],
                              idd_v.at[pl.ds(slot, 1)], iqs[slot]).wait()

    def issue_gather(slot, idx_slot):
        pltpu.async_copy(tab_hbm.at[ids_v.at[idx_slot]], rows_v.at[slot],
                         gss[slot])

    def wait_gather(slot):
        pltpu.make_async_copy(tab_hbm.at[pl.ds(0, chunk)], rows_v.at[slot],
                              gss[slot]).wait()

    def issue_scatter(slot, idx_slot):
        pltpu.async_copy(rows_v.at[slot], acc_sh.at[idd_v.at[idx_slot]],
                         sss[slot], add=True)

    def wait_scatter(slot):
        pltpu.make_async_copy(rows_v.at[slot], acc_sh.at[idd_v.at[0]],
                              sss[slot]).wait()

    for q in range(6):
        issue_idx(q, q)
    wait_idx(0)
    wait_idx(1)
    pltpu.sync_copy(z_hbm.at[pl.ds(0, chunk)], rows_v.at[2])
    pltpu.sync_copy(z_hbm.at[pl.ds(0, chunk)], rows_v.at[3])
    issue_scatter(2, 0)
    issue_scatter(3, 1)
    issue_gather(0, 0)
    issue_gather(1, 1)

    def body(g, carry):
        for b in range(8):
            i = g * 8 + b
            r = b % 4
            wait_gather(r)
            issue_scatter(r, b)
            wait_scatter((r + 2) % 4)
            wait_idx((b + 2) % 8)
            issue_gather((r + 2) % 4, (b + 2) % 8)
            issue_idx((b + 6) % 8, jnp.minimum(i + 6, nch - 1))
        return carry

    lax.fori_loop(0, nch // 8, body, 0)
    for q in (2, 3, 4, 5):
        wait_idx(q)
    for r in (0, 1):
        wait_gather(r)
    for r in (2, 3):
        wait_scatter(r)


def _agg128_sc(table, src2d, dst2d, zeros64):
    ch = 64
    nch = EPAD // NS // ch

    @functools.partial(
        pl.kernel,
        out_type=jax.ShapeDtypeStruct((2 * NPAD, 128), _f32),
        mesh=_sc_mesh(),
        compiler_params=pltpu.CompilerParams(use_tc_tiling_on_sc=False),
        scratch_types=(
            [pltpu.VMEM((8, ch), jnp.int32),
             pltpu.VMEM((8, ch), jnp.int32),
             pltpu.VMEM((4, ch, 64), _f32),
             pltpu.VMEM_SHARED((NPAD, 64), _f32),
             pltpu.VMEM_SHARED((NPAD, 64), _f32)]
            + [pltpu.SemaphoreType.DMA] * 16
        ),
    )
    def k(tab_hbm, src_hbm, dst_hbm, z_hbm, out_hbm, ids_v, idd_v, rows_v,
          tab_sh, acc_sh, *sems):
        c = lax.axis_index("c")
        s = lax.axis_index("s")
        rows0 = s * RPT
        for p in (0, 1):
            pltpu.sync_copy(
                tab_hbm.at[pl.ds(c * NPAD + rows0, RPT), pl.ds(p * 64, 64)],
                tab_sh.at[pl.ds(rows0, RPT)])
            pltpu.sync_copy(z_hbm.at[pl.ds(rows0, RPT)],
                            acc_sh.at[pl.ds(rows0, RPT)])
            plsc.subcore_barrier()
            _agg_pipe(tab_sh, src_hbm, dst_hbm, z_hbm, acc_sh, ids_v, idd_v,
                      rows_v, sems[:8], sems[8:12], sems[12:16],
                      s * nch, s * nch, nch, ch)
            plsc.subcore_barrier()
            pltpu.sync_copy(
                acc_sh.at[pl.ds(rows0, RPT)],
                out_hbm.at[pl.ds(c * NPAD + rows0, RPT), pl.ds(p * 64, 64)])
            plsc.subcore_barrier()

    return k(table, src2d, dst2d, zeros64)


def _agg16_sc(table, src2d, dst2d, zeros16):
    nch = EPAD // (NC * NS) // CHUNK

    @functools.partial(
        pl.kernel,
        out_type=jax.ShapeDtypeStruct((2 * NPAD, 16), _f32),
        mesh=_sc_mesh(),
        compiler_params=pltpu.CompilerParams(use_tc_tiling_on_sc=False),
        scratch_types=(
            [pltpu.VMEM((8, CHUNK), jnp.int32),
             pltpu.VMEM((8, CHUNK), jnp.int32),
             pltpu.VMEM((4, CHUNK, 16), _f32),
             pltpu.VMEM_SHARED((NPAD, 16), _f32),
             pltpu.VMEM_SHARED((NPAD, 16), _f32)]
            + [pltpu.SemaphoreType.DMA] * 16
        ),
    )
    def k(tab_hbm, src_hbm, dst_hbm, z_hbm, out_hbm, ids_v, idd_v, rows_v,
          tab_sh, acc_sh, *sems):
        c = lax.axis_index("c")
        s = lax.axis_index("s")
        rows0 = s * RPT
        wid = s * NC + c
        pltpu.sync_copy(tab_hbm.at[pl.ds(rows0, RPT)],
                        tab_sh.at[pl.ds(rows0, RPT)])
        pltpu.sync_copy(z_hbm.at[pl.ds(rows0, RPT)], acc_sh.at[pl.ds(rows0, RPT)])
        plsc.subcore_barrier()
        _agg_pipe(tab_sh, src_hbm, dst_hbm, z_hbm, acc_sh, ids_v, idd_v,
                  rows_v, sems[:8], sems[8:12], sems[12:16],
                  wid * nch, wid * nch, nch, CHUNK)
        plsc.subcore_barrier()
        pltpu.sync_copy(acc_sh.at[pl.ds(rows0, RPT)],
                        out_hbm.at[pl.ds(c * NPAD + rows0, RPT)])

    return k(table, src2d, dst2d, zeros16)


def _k1_body(x_ref, w_ref, dega_ref, degb_ref, g1s_ref, dis_ref):
    deg = dega_ref[:, :1] + degb_ref[:, :1] + 1.0
    dis = lax.rsqrt(deg)
    g = jnp.dot(x_ref[...], w_ref[...], preferred_element_type=_f32)
    g1s_ref[...] = g * dis
    dis_ref[...] = jnp.broadcast_to(dis, (BLK, 16))


def _k1(xp, W1, deg2):
    return pl.pallas_call(
        _k1_body,
        grid=(NB, 2),
        in_specs=[
            pl.BlockSpec((BLK, 128), lambda i, j: (i, 0)),
            pl.BlockSpec((128, 128), lambda i, j: (0, j)),
            pl.BlockSpec((BLK, 16), lambda i, j: (i, 0)),
            pl.BlockSpec((BLK, 16), lambda i, j: (i + NB, 0)),
        ],
        out_specs=[
            pl.BlockSpec((BLK, 128), lambda i, j: (j * NB + i, 0)),
            pl.BlockSpec((BLK, 16), lambda i, j: (i, 0)),
        ],
        out_shape=[
            jax.ShapeDtypeStruct((2 * NPAD, 128), _f32),
            jax.ShapeDtypeStruct((NPAD, 16), _f32),
        ],
    )(xp, W1, deg2, deg2)


def _k2_body(slo_ref, shi_ref, glo_ref, ghi_ref, dis_ref, b_ref, wa_ref,
             wb_ref, out_ref):
    dis = dis_ref[:, :1]
    hlo = jnp.maximum(dis * (slo_ref[...] + glo_ref[...]) + b_ref[0:1, :], 0.0)
    hhi = jnp.maximum(dis * (shi_ref[...] + ghi_ref[...]) + b_ref[1:2, :], 0.0)
    g2 = (jnp.dot(hlo, wa_ref[...], preferred_element_type=_f32)
          + jnp.dot(hhi, wb_ref[...], preferred_element_type=_f32))
    out_ref[...] = dis * g2


def _k2(s1, g1s, dis, b1r, W2):
    return pl.pallas_call(
        _k2_body,
        grid=(NB, 2),
        in_specs=[
            pl.BlockSpec((BLK, 128), lambda i, j: (i, 0)),
            pl.BlockSpec((BLK, 128), lambda i, j: (i + NB, 0)),
            pl.BlockSpec((BLK, 128), lambda i, j: (i, 0)),
            pl.BlockSpec((BLK, 128), lambda i, j: (i + NB, 0)),
            pl.BlockSpec((BLK, 16), lambda i, j: (i, 0)),
            pl.BlockSpec((2, 128), lambda i, j: (0, 0)),
            pl.BlockSpec((128, 128), lambda i, j: (0, j)),
            pl.BlockSpec((128, 128), lambda i, j: (1, j)),
        ],
        out_specs=pl.BlockSpec((BLK, 128), lambda i, j: (j * NB + i, 0)),
        out_shape=jax.ShapeDtypeStruct((2 * NPAD, 128), _f32),
    )(s1, s1, g1s, g1s, dis, b1r, W2, W2)


def _k3_body(slo_ref, shi_ref, glo_ref, ghi_ref, dis_ref, b_ref, wa_ref,
             wb_ref, out_ref):
    dis = dis_ref[:, :1]
    hlo = jnp.maximum(dis * (slo_ref[...] + glo_ref[...]) + b_ref[0:1, :], 0.0)
    hhi = jnp.maximum(dis * (shi_ref[...] + ghi_ref[...]) + b_ref[1:2, :], 0.0)
    g3 = (jnp.dot(hlo, wa_ref[...], preferred_element_type=_f32)
          + jnp.dot(hhi, wb_ref[...], preferred_element_type=_f32))
    out_ref[...] = dis * g3


def _k3(s2, g2s, dis, b2r, W316):
    return pl.pallas_call(
        _k3_body,
        grid=(NB,),
        in_specs=[
            pl.BlockSpec((BLK, 128), lambda i: (i, 0)),
            pl.BlockSpec((BLK, 128), lambda i: (i + NB, 0)),
            pl.BlockSpec((BLK, 128), lambda i: (i, 0)),
            pl.BlockSpec((BLK, 128), lambda i: (i + NB, 0)),
            pl.BlockSpec((BLK, 16), lambda i: (i, 0)),
            pl.BlockSpec((2, 128), lambda i: (0, 0)),
            pl.BlockSpec((128, 16), lambda i: (0, 0)),
            pl.BlockSpec((128, 16), lambda i: (1, 0)),
        ],
        out_specs=pl.BlockSpec((BLK, 16), lambda i: (i, 0)),
        out_shape=jax.ShapeDtypeStruct((NPAD, 16), _f32),
    )(s2, s2, g2s, g2s, dis, b2r, W316, W316)


def _k4_body(s3a_ref, s3b_ref, g3s_ref, dis_ref, batch_ref, b3_ref, out_ref):
    s3 = s3a_ref[...] + s3b_ref[...]
    y = dis_ref[...] * (s3 + g3s_ref[...]) + b3_ref[...]
    gids = lax.broadcasted_iota(jnp.int32, (1, 16), 1)
    onehot = (batch_ref[...] == gids).astype(_f32)
    sums = jnp.sum(onehot * y, axis=0)
    counts = jnp.sum(onehot, axis=0)
    out_ref[...] = (sums / jnp.maximum(counts, 1.0))[None, :]


def _k4(s3, g3s16, dis, batchi, b3r):
    return pl.pallas_call(
        _k4_body,
        grid=(1,),
        in_specs=[
            pl.BlockSpec((NPAD, 16), lambda i: (0, 0)),
            pl.BlockSpec((NPAD, 16), lambda i: (1, 0)),
            pl.BlockSpec((NPAD, 16), lambda i: (0, 0)),
            pl.BlockSpec((NPAD, 16), lambda i: (0, 0)),
            pl.BlockSpec((NPAD, 16), lambda i: (0, 0)),
            pl.BlockSpec((1, 16), lambda i: (0, 0)),
        ],
        out_specs=pl.BlockSpec((1, 16), lambda i: (0, 0)),
        out_shape=jax.ShapeDtypeStruct((1, 16), _f32),
    )(s3, s3, g3s16, dis, batchi, b3r)


def kernel(x, edge_index, batch, W1, b1, W2, b2, W3, b3):
    pad = jnp.full((EPAD - E,), NPAD - 1, dtype=jnp.int32)
    srcp = jnp.concatenate([edge_index[0], pad])
    dstp = jnp.concatenate([edge_index[1], pad])
    src2d = srcp.reshape(EPAD // CHUNK, CHUNK)
    src2d_64 = srcp.reshape(EPAD // 64, 64)
    dst2d = dstp.reshape(EPAD // CHUNK, CHUNK)
    dst2d_64 = dstp.reshape(EPAD // 64, 64)
    xp = jnp.pad(x, ((0, NPAD - N), (0, 0)))
    batchp = jnp.pad(batch, (0, NPAD - N), constant_values=NUM_GRAPHS)
    batchi = jnp.broadcast_to(batchp[:, None], (NPAD, 16))
    b1r = b1.reshape(2, 128)
    b2r = b2.reshape(2, 128)
    W316 = jnp.tile(W3, (1, 16))
    b3r = jnp.broadcast_to(b3.reshape(1, 1), (1, 16))
    zeros64 = jnp.zeros((NPAD, 64), _f32)
    zeros16 = jnp.zeros((NPAD, 16), _f32)
    ones16 = jnp.ones((CHUNK, 16), _f32)

    deg2 = _deg_sc(dst2d, zeros16, ones16)
    g1s, dis = _k1(xp, W1, deg2)
    s1 = _agg128_sc(g1s, src2d_64, dst2d_64, zeros64)
    g2s = _k2(s1, g1s, dis, b1r, W2)
    s2 = _agg128_sc(g2s, src2d_64, dst2d_64, zeros64)
    g3s16 = _k3(s2, g2s, dis, b2r, W316)
    s3 = _agg16_sc(g3s16, src2d, dst2d, zeros16)
    out = _k4(s3, g3s16, dis, batchi, b3r)
    return out.reshape(NUM_GRAPHS, 1)
